# Initial kernel scaffold; baseline (speedup 1.0000x reference)
import functools

import jax
import jax.numpy as jnp
from jax import lax
from jax.experimental import pallas as pl
from jax.experimental.pallas import tpu as pltpu
from jax.experimental.pallas import tpu_sc as plsc

N = 16384
T = 50
ED = 32
NCAT = 100

NC = 2
NS = 16
NW = NC * NS
RPW = N // NW
L = 16


def _make_trait_pool():
    mesh = plsc.VectorSubcoreMesh(core_axis_name="c", subcore_axis_name="s")

    @functools.partial(
        pl.kernel,
        mesh=mesh,
        out_type=jax.ShapeDtypeStruct((N, ED), jnp.float32),
        scratch_types=[
            pltpu.VMEM((RPW * T,), jnp.int32),
            pltpu.VMEM((RPW * T,), jnp.float32),
            pltpu.VMEM((T, ED), jnp.float32),
            pltpu.VMEM((T, ED), jnp.float32),
            pltpu.VMEM((RPW, ED), jnp.float32),
            pltpu.SemaphoreType.DMA,
            pltpu.SemaphoreType.DMA,
        ],
    )
    def trait_pool(ids_hbm, scores_hbm, table_hbm, out_hbm,
                   ids_v, w_v, buf_a, buf_b, out_v, sem_a, sem_b):
        wid = lax.axis_index("s") * NC + lax.axis_index("c")
        base = wid * (RPW * T)

        pltpu.sync_copy(ids_hbm.at[pl.ds(base, RPW * T)], ids_v)
        pltpu.sync_copy(scores_hbm.at[pl.ds(base, RPW * T)], w_v)

        def mask_body(k, carry):
            sl = pl.ds(k * L, L)
            w_v[sl] = jnp.where(ids_v[sl] != 0, w_v[sl], 0.0)
            return carry

        lax.fori_loop(0, (RPW * T) // L, mask_body, 0)

        def start(r, buf, sem):
            return pltpu.async_copy(
                table_hbm.at[ids_v.at[pl.ds(r * T, T)]], buf, sem)

        def wait(buf, sem):
            pltpu.make_async_copy(
                table_hbm.at[ids_v.at[pl.ds(0, T)]], buf, sem).wait()

        def accum_row(r, buf):
            acc0 = jnp.zeros((L,), jnp.float32)
            acc1 = jnp.zeros((L,), jnp.float32)
            for t in range(T):
                wv = plsc.load_gather(
                    w_v, [jnp.full((L,), r * T + t, jnp.int32)])
                acc0 = acc0 + wv * buf[t, pl.ds(0, L)]
                acc1 = acc1 + wv * buf[t, pl.ds(L, L)]
            out_v[r, pl.ds(0, L)] = acc0
            out_v[r, pl.ds(L, L)] = acc1

        start(0, buf_a, sem_a)

        def pair_body(p, carry):
            r0 = 2 * p
            start(r0 + 1, buf_b, sem_b)
            wait(buf_a, sem_a)
            accum_row(r0, buf_a)

            @pl.when(p + 1 < RPW // 2)
            def _prefetch():
                start(r0 + 2, buf_a, sem_a)

            wait(buf_b, sem_b)
            accum_row(r0 + 1, buf_b)
            return carry

        lax.fori_loop(0, RPW // 2, pair_body, 0)

        pltpu.sync_copy(out_v, out_hbm.at[pl.ds(wid * RPW, RPW)])

    return trait_pool


_trait_pool = _make_trait_pool()


BLK = 128


def _finish_body(g_ref, ids_ref, sc_ref, cat_ref, ctab_ref, w_ref, b_ref,
                 out_ref):
    ids = ids_ref[...]
    s = sc_ref[...]
    w = jnp.where(ids != 0, s, 0.0)
    d = jnp.sum(w, axis=1, keepdims=True)
    a = jnp.sum(w * s, axis=1, keepdims=True)

    cat = cat_ref[...]
    cvals = lax.broadcasted_iota(jnp.int32, (1, 1, NCAT), 2)
    onehot = jnp.where(cat[:, :, None] == cvals, w[:, :, None], 0.0)
    m = jnp.sum(onehot, axis=1)
    catp = jnp.dot(m, ctab_ref[...], preferred_element_type=jnp.float32)

    x = jnp.concatenate([g_ref[...], catp, a], axis=1)
    num = jnp.dot(x, w_ref[...], preferred_element_type=jnp.float32)
    num = num + d * b_ref[...]
    out_ref[...] = num / jnp.maximum(d, 1e-8)


def _finish(g, ids, scores, cat_ids, cat_table, W, b2d):
    grid = (N // BLK,)
    return pl.pallas_call(
        _finish_body,
        grid=grid,
        in_specs=[
            pl.BlockSpec((BLK, ED), lambda i: (i, 0)),
            pl.BlockSpec((BLK, T), lambda i: (i, 0)),
            pl.BlockSpec((BLK, T), lambda i: (i, 0)),
            pl.BlockSpec((BLK, T), lambda i: (i, 0)),
            pl.BlockSpec((NCAT, 8), lambda i: (0, 0)),
            pl.BlockSpec((ED + 9, ED), lambda i: (0, 0)),
            pl.BlockSpec((1, ED), lambda i: (0, 0)),
        ],
        out_specs=pl.BlockSpec((BLK, ED), lambda i: (i, 0)),
        out_shape=jax.ShapeDtypeStruct((N, ED), jnp.float32),
    )(g, ids, scores, cat_ids, cat_table, W, b2d)


def kernel(token_ids, scores, cat_ids, trait_table, cat_table, W, b):
    ids = token_ids.astype(jnp.int32)
    cids = cat_ids.astype(jnp.int32)
    g = _trait_pool(ids.reshape(-1), scores.reshape(-1), trait_table)
    return _finish(g, ids, scores, cids, cat_table, W, b.reshape(1, ED))


# speedup vs baseline: 23.3767x; 23.3767x over previous
import functools

import jax
import jax.numpy as jnp
from jax import lax
from jax.experimental import pallas as pl
from jax.experimental.pallas import tpu as pltpu
from jax.experimental.pallas import tpu_sc as plsc

N = 16384
T = 50
TP = 64
ED = 32
NCAT = 100

NC = 2
NS = 16
NW = NC * NS
RPW = N // NW
L = 16


def _make_trait_pool():
    mesh = plsc.VectorSubcoreMesh(core_axis_name="c", subcore_axis_name="s")

    @functools.partial(
        pl.kernel,
        mesh=mesh,
        out_type=jax.ShapeDtypeStruct((N, ED), jnp.float32),
        compiler_params=pltpu.CompilerParams(use_tc_tiling_on_sc=False),
        scratch_types=[
            pltpu.VMEM((RPW * TP,), jnp.int32),
            pltpu.VMEM((RPW, T), jnp.int32),
            pltpu.VMEM((RPW * TP,), jnp.float32),
            pltpu.VMEM((T, ED), jnp.float32),
            pltpu.VMEM((T, ED), jnp.float32),
            pltpu.VMEM((RPW, ED), jnp.float32),
            pltpu.SemaphoreType.DMA,
            pltpu.SemaphoreType.DMA,
        ],
    )
    def trait_pool(ids_hbm, ids2_hbm, scores_hbm, table_hbm, out_hbm,
                   ids_v, ids2_v, w_v, buf_a, buf_b, out_v, sem_a, sem_b):
        wid = lax.axis_index("s") * NC + lax.axis_index("c")
        base = wid * (RPW * TP)

        pltpu.sync_copy(ids_hbm.at[pl.ds(base, RPW * TP)], ids_v)
        pltpu.sync_copy(ids2_hbm.at[pl.ds(wid * RPW, RPW), :], ids2_v)
        pltpu.sync_copy(scores_hbm.at[pl.ds(base, RPW * TP)], w_v)

        def mask_body(k, carry):
            sl = pl.ds(k * L, L)
            w_v[sl] = jnp.where(ids_v[sl] != 0, w_v[sl], 0.0)
            return carry

        lax.fori_loop(0, (RPW * TP) // L, mask_body, 0)

        def start(r, buf, sem):
            return pltpu.async_copy(
                table_hbm.at[ids2_v.at[r]], buf, sem)

        def wait(buf, sem):
            pltpu.make_async_copy(
                table_hbm.at[ids2_v.at[0]], buf, sem).wait()

        def accum_row(r, buf):
            acc0 = jnp.zeros((L,), jnp.float32)
            acc1 = jnp.zeros((L,), jnp.float32)
            for ck in range(T // L + 1):
                wchunk = w_v[pl.ds(r * TP + ck * L, L)]
                for t in range(min(L, T - ck * L)):
                    wv = jnp.full((L,), wchunk[t], jnp.float32)
                    tok = ck * L + t
                    acc0 = acc0 + wv * buf[tok, pl.ds(0, L)]
                    acc1 = acc1 + wv * buf[tok, pl.ds(L, L)]
            out_v[r, pl.ds(0, L)] = acc0
            out_v[r, pl.ds(L, L)] = acc1

        start(0, buf_a, sem_a)

        def pair_body(p, carry):
            r0 = 2 * p
            start(r0 + 1, buf_b, sem_b)
            wait(buf_a, sem_a)
            accum_row(r0, buf_a)

            @pl.when(p + 1 < RPW // 2)
            def _prefetch():
                start(r0 + 2, buf_a, sem_a)

            wait(buf_b, sem_b)
            accum_row(r0 + 1, buf_b)
            return carry

        lax.fori_loop(0, RPW // 2, pair_body, 0)

        pltpu.sync_copy(out_v, out_hbm.at[pl.ds(wid * RPW, RPW)])

    return trait_pool


_trait_pool = _make_trait_pool()


BLK = 128


def _finish_body(g_ref, ids_ref, sc_ref, cat_ref, ctab_ref, w_ref, b_ref,
                 out_ref):
    ids = ids_ref[...]
    s = sc_ref[...]
    w = jnp.where(ids != 0, s, 0.0)
    d = jnp.sum(w, axis=1, keepdims=True)
    a = jnp.sum(w * s, axis=1, keepdims=True)

    cat = cat_ref[...]
    cvals = lax.broadcasted_iota(jnp.int32, (1, 1, NCAT), 2)
    onehot = jnp.where(cat[:, :, None] == cvals, w[:, :, None], 0.0)
    m = jnp.sum(onehot, axis=1)
    catp = jnp.dot(m, ctab_ref[...], preferred_element_type=jnp.float32)

    x = jnp.concatenate([g_ref[...], catp, a], axis=1)
    num = jnp.dot(x, w_ref[...], preferred_element_type=jnp.float32)
    num = num + d * b_ref[...]
    out_ref[...] = num / jnp.maximum(d, 1e-8)


def _finish(g, ids, scores, cat_ids, cat_table, W, b2d):
    grid = (N // BLK,)
    return pl.pallas_call(
        _finish_body,
        grid=grid,
        in_specs=[
            pl.BlockSpec((BLK, ED), lambda i: (i, 0)),
            pl.BlockSpec((BLK, T), lambda i: (i, 0)),
            pl.BlockSpec((BLK, T), lambda i: (i, 0)),
            pl.BlockSpec((BLK, T), lambda i: (i, 0)),
            pl.BlockSpec((NCAT, 8), lambda i: (0, 0)),
            pl.BlockSpec((ED + 9, ED), lambda i: (0, 0)),
            pl.BlockSpec((1, ED), lambda i: (0, 0)),
        ],
        out_specs=pl.BlockSpec((BLK, ED), lambda i: (i, 0)),
        out_shape=jax.ShapeDtypeStruct((N, ED), jnp.float32),
    )(g, ids, scores, cat_ids, cat_table, W, b2d)


def kernel(token_ids, scores, cat_ids, trait_table, cat_table, W, b):
    ids = token_ids.astype(jnp.int32)
    cids = cat_ids.astype(jnp.int32)
    ids_p = jnp.pad(ids, ((0, 0), (0, TP - T))).reshape(-1)
    sc_p = jnp.pad(scores, ((0, 0), (0, TP - T))).reshape(-1)
    g = _trait_pool(ids_p, ids, sc_p, trait_table)
    return _finish(g, ids, scores, cids, cat_table, W, b.reshape(1, ED))


# speedup vs baseline: 30.9951x; 1.3259x over previous
import functools

import jax
import jax.numpy as jnp
from jax import lax
from jax.experimental import pallas as pl
from jax.experimental.pallas import tpu as pltpu
from jax.experimental.pallas import tpu_sc as plsc

N = 16384
T = 50
TP = 64
ED = 32
NCAT = 100
CP = 16
PD = ED + CP

NC = 2
NS = 16
NW = NC * NS
RPW = N // NW
L = 16


def _make_pool():
    mesh = plsc.VectorSubcoreMesh(core_axis_name="c", subcore_axis_name="s")

    @functools.partial(
        pl.kernel,
        mesh=mesh,
        out_type=jax.ShapeDtypeStruct((N, PD), jnp.float32),
        compiler_params=pltpu.CompilerParams(use_tc_tiling_on_sc=False),
        scratch_types=[
            pltpu.VMEM((RPW * TP,), jnp.int32),
            pltpu.VMEM((RPW, T), jnp.int32),
            pltpu.VMEM((RPW * TP,), jnp.float32),
            pltpu.VMEM((NCAT, CP), jnp.float32),
            pltpu.VMEM((T, ED), jnp.float32),
            pltpu.VMEM((T, ED), jnp.float32),
            pltpu.VMEM((RPW, PD), jnp.float32),
            pltpu.SemaphoreType.DMA,
            pltpu.SemaphoreType.DMA,
        ],
    )
    def pool(ids_hbm, ids2_hbm, cids_hbm, scores_hbm, table_hbm, ctab_hbm,
             out_hbm, ids_v, ids2_v, w_v, ctab_v, buf_a, buf_b, out_v,
             sem_a, sem_b):
        wid = lax.axis_index("s") * NC + lax.axis_index("c")
        base = wid * (RPW * TP)

        pltpu.sync_copy(ids_hbm.at[pl.ds(base, RPW * TP)], ids_v)
        pltpu.sync_copy(ids2_hbm.at[pl.ds(wid * RPW, RPW), :], ids2_v)
        pltpu.sync_copy(scores_hbm.at[pl.ds(base, RPW * TP)], w_v)
        pltpu.sync_copy(ctab_hbm, ctab_v)

        def mask_body(k, carry):
            sl = pl.ds(k * L, L)
            w_v[sl] = jnp.where(ids_v[sl] != 0, w_v[sl], 0.0)
            return carry

        lax.fori_loop(0, (RPW * TP) // L, mask_body, 0)

        pltpu.sync_copy(cids_hbm.at[pl.ds(base, RPW * TP)], ids_v)

        def start(r, buf, sem):
            return pltpu.async_copy(
                table_hbm.at[ids2_v.at[r]], buf, sem)

        def wait(buf, sem):
            pltpu.make_async_copy(
                table_hbm.at[ids2_v.at[0]], buf, sem).wait()

        def accum_row(r, buf):
            acc0 = jnp.zeros((L,), jnp.float32)
            acc1 = jnp.zeros((L,), jnp.float32)
            acc2 = jnp.zeros((L,), jnp.float32)
            for ck in range(T // L + 1):
                wchunk = w_v[pl.ds(r * TP + ck * L, L)]
                cchunk = ids_v[pl.ds(r * TP + ck * L, L)]
                for t in range(min(L, T - ck * L)):
                    wv = jnp.full((L,), wchunk[t], jnp.float32)
                    tok = ck * L + t
                    acc0 = acc0 + wv * buf[tok, pl.ds(0, L)]
                    acc1 = acc1 + wv * buf[tok, pl.ds(L, L)]
                    acc2 = acc2 + wv * ctab_v[cchunk[t], pl.ds(0, L)]
            out_v[r, pl.ds(0, L)] = acc0
            out_v[r, pl.ds(L, L)] = acc1
            out_v[r, pl.ds(2 * L, L)] = acc2

        start(0, buf_a, sem_a)

        def pair_body(p, carry):
            r0 = 2 * p
            start(r0 + 1, buf_b, sem_b)
            wait(buf_a, sem_a)
            accum_row(r0, buf_a)

            @pl.when(p + 1 < RPW // 2)
            def _prefetch():
                start(r0 + 2, buf_a, sem_a)

            wait(buf_b, sem_b)
            accum_row(r0 + 1, buf_b)
            return carry

        lax.fori_loop(0, RPW // 2, pair_body, 0)

        pltpu.sync_copy(out_v, out_hbm.at[pl.ds(wid * RPW, RPW)])

    return pool


_pool = _make_pool()


BLK = 512


def _finish_body(p_ref, ids_ref, sc_ref, w_ref, b_ref, out_ref):
    ids = ids_ref[...]
    s = sc_ref[...]
    w = jnp.where(ids != 0, s, 0.0)
    d = jnp.sum(w, axis=1, keepdims=True)
    a = jnp.sum(w * s, axis=1, keepdims=True)

    x = jnp.concatenate([p_ref[:, : ED + 8], a], axis=1)
    num = jnp.dot(x, w_ref[...], preferred_element_type=jnp.float32)
    num = num + d * b_ref[...]
    out_ref[...] = num / jnp.maximum(d, 1e-8)


def _finish(pooled, ids, scores, W, b2d):
    grid = (N // BLK,)
    return pl.pallas_call(
        _finish_body,
        grid=grid,
        in_specs=[
            pl.BlockSpec((BLK, PD), lambda i: (i, 0)),
            pl.BlockSpec((BLK, T), lambda i: (i, 0)),
            pl.BlockSpec((BLK, T), lambda i: (i, 0)),
            pl.BlockSpec((ED + 9, ED), lambda i: (0, 0)),
            pl.BlockSpec((1, ED), lambda i: (0, 0)),
        ],
        out_specs=pl.BlockSpec((BLK, ED), lambda i: (i, 0)),
        out_shape=jax.ShapeDtypeStruct((N, ED), jnp.float32),
    )(pooled, ids, scores, W, b2d)


def kernel(token_ids, scores, cat_ids, trait_table, cat_table, W, b):
    ids = token_ids.astype(jnp.int32)
    cids = cat_ids.astype(jnp.int32)
    ids_p = jnp.pad(ids, ((0, 0), (0, TP - T))).reshape(-1)
    cids_p = jnp.pad(cids, ((0, 0), (0, TP - T))).reshape(-1)
    sc_p = jnp.pad(scores, ((0, 0), (0, TP - T))).reshape(-1)
    ctab16 = jnp.pad(cat_table, ((0, 0), (0, CP - 8)))
    pooled = _pool(ids_p, ids, cids_p, sc_p, trait_table, ctab16)
    return _finish(pooled, ids, scores, W, b.reshape(1, ED))


# speedup vs baseline: 31.5051x; 1.0165x over previous
import functools

import jax
import jax.numpy as jnp
from jax import lax
from jax.experimental import pallas as pl
from jax.experimental.pallas import tpu as pltpu
from jax.experimental.pallas import tpu_sc as plsc

N = 16384
T = 50
ED = 32
NCAT = 100
CP = 16
PD = ED + CP

NC = 2
NS = 16
NW = NC * NS
RPW = N // NW
L = 16
NCK = T // L + 1
ROWS_S = RPW + 1


def _make_pool():
    mesh = plsc.VectorSubcoreMesh(core_axis_name="c", subcore_axis_name="s")

    @functools.partial(
        pl.kernel,
        mesh=mesh,
        out_type=jax.ShapeDtypeStruct((N, PD), jnp.float32),
        compiler_params=pltpu.CompilerParams(use_tc_tiling_on_sc=False),
        scratch_types=[
            pltpu.VMEM((ROWS_S, T), jnp.int32),
            pltpu.VMEM((ROWS_S, T), jnp.int32),
            pltpu.VMEM((ROWS_S, T), jnp.float32),
            pltpu.VMEM((NCAT, CP), jnp.float32),
            pltpu.VMEM((T, ED), jnp.float32),
            pltpu.VMEM((T, ED), jnp.float32),
            pltpu.VMEM((RPW, PD), jnp.float32),
            pltpu.SemaphoreType.DMA,
            pltpu.SemaphoreType.DMA,
        ],
    )
    def pool(ids_hbm, cids_hbm, scores_hbm, table_hbm, ctab_hbm,
             out_hbm, ids_v, cids_v, w_v, ctab_v, buf_a, buf_b, out_v,
             sem_a, sem_b):
        wid = lax.axis_index("s") * NC + lax.axis_index("c")
        base = wid * RPW

        pltpu.sync_copy(ids_hbm.at[pl.ds(base, RPW), :],
                        ids_v.at[pl.ds(0, RPW), :])
        pltpu.sync_copy(cids_hbm.at[pl.ds(base, RPW), :],
                        cids_v.at[pl.ds(0, RPW), :])
        pltpu.sync_copy(scores_hbm.at[pl.ds(base, RPW), :],
                        w_v.at[pl.ds(0, RPW), :])
        pltpu.sync_copy(ctab_hbm, ctab_v)

        def mask_body(r, carry):
            for ck in range(NCK):
                sl = pl.ds(min(ck * L, T - L), L)
                w_v[r, sl] = jnp.where(ids_v[r, sl] != 0, w_v[r, sl], 0.0)
            return carry

        lax.fori_loop(0, RPW, mask_body, 0)

        def start(r, buf, sem):
            return pltpu.async_copy(
                table_hbm.at[ids_v.at[r]], buf, sem)

        def wait(buf, sem):
            pltpu.make_async_copy(
                table_hbm.at[ids_v.at[0]], buf, sem).wait()

        def accum_row(r, buf):
            acc0 = jnp.zeros((L,), jnp.float32)
            acc1 = jnp.zeros((L,), jnp.float32)
            acc2 = jnp.zeros((L,), jnp.float32)
            for ck in range(NCK):
                off = min(ck * L, T - L)
                wchunk = w_v[r, pl.ds(off, L)]
                cchunk = cids_v[r, pl.ds(off, L)]
                for t in range(0 if ck < NCK - 1 else (NCK - 1) * L - off,
                               L):
                    wv = jnp.full((L,), wchunk[t], jnp.float32)
                    tok = off + t
                    acc0 = acc0 + wv * buf[tok, pl.ds(0, L)]
                    acc1 = acc1 + wv * buf[tok, pl.ds(L, L)]
                    acc2 = acc2 + wv * ctab_v[cchunk[t], pl.ds(0, L)]
            out_v[r, pl.ds(0, L)] = acc0
            out_v[r, pl.ds(L, L)] = acc1
            out_v[r, pl.ds(2 * L, L)] = acc2

        start(0, buf_a, sem_a)

        def pair_body(p, carry):
            r0 = 2 * p
            start(r0 + 1, buf_b, sem_b)
            wait(buf_a, sem_a)
            accum_row(r0, buf_a)

            @pl.when(p + 1 < RPW // 2)
            def _prefetch():
                start(r0 + 2, buf_a, sem_a)

            wait(buf_b, sem_b)
            accum_row(r0 + 1, buf_b)
            return carry

        lax.fori_loop(0, RPW // 2, pair_body, 0)

        pltpu.sync_copy(out_v, out_hbm.at[pl.ds(base, RPW)])

    return pool


_pool = _make_pool()


BLK = 512


def _finish_body(p_ref, ids_ref, sc_ref, w_ref, b_ref, out_ref):
    ids = ids_ref[...]
    s = sc_ref[...]
    w = jnp.where(ids != 0, s, 0.0)
    d = jnp.sum(w, axis=1, keepdims=True)
    a = jnp.sum(w * s, axis=1, keepdims=True)

    x = jnp.concatenate([p_ref[:, : ED + 8], a], axis=1)
    num = jnp.dot(x, w_ref[...], preferred_element_type=jnp.float32)
    num = num + d * b_ref[...]
    out_ref[...] = num / jnp.maximum(d, 1e-8)


def _finish(pooled, ids, scores, W, b2d):
    grid = (N // BLK,)
    return pl.pallas_call(
        _finish_body,
        grid=grid,
        in_specs=[
            pl.BlockSpec((BLK, PD), lambda i: (i, 0)),
            pl.BlockSpec((BLK, T), lambda i: (i, 0)),
            pl.BlockSpec((BLK, T), lambda i: (i, 0)),
            pl.BlockSpec((ED + 9, ED), lambda i: (0, 0)),
            pl.BlockSpec((1, ED), lambda i: (0, 0)),
        ],
        out_specs=pl.BlockSpec((BLK, ED), lambda i: (i, 0)),
        out_shape=jax.ShapeDtypeStruct((N, ED), jnp.float32),
    )(pooled, ids, scores, W, b2d)


def kernel(token_ids, scores, cat_ids, trait_table, cat_table, W, b):
    ids = token_ids.astype(jnp.int32)
    cids = cat_ids.astype(jnp.int32)
    ctab16 = jnp.pad(cat_table, ((0, 0), (0, CP - 8)))
    pooled = _pool(ids, cids, scores, trait_table, ctab16)
    return _finish(pooled, ids, scores, W, b.reshape(1, ED))
